# baseline (speedup 1.0000x reference)
import functools

import jax
import jax.numpy as jnp
import numpy as np
from jax import lax
from jax.experimental import pallas as pl
from jax.experimental.pallas import tpu as pltpu
from jax.experimental.pallas import tpu_sc as plsc

_NC = 19
_EPS = float(np.finfo(np.float32).eps)

_NPIX = 512 * 512
_NB = 4
_Q_TC = 131072
_TC_CHUNK = 32768
_NW = 32
_SC_CHUNK = 2048


def _tc_body(x_ref, t_ref, hist_ref, acc_ref, *, num_steps, chunk):
    step = pl.program_id(0)

    @pl.when(step == 0)
    def _init():
        acc_ref[...] = jnp.zeros_like(acc_ref)

    x = x_ref[0]
    t = t_ref[0]
    cls = lax.broadcasted_iota(jnp.int32, (_NC, chunk), 0)
    m = jnp.max(x, axis=0, keepdims=True)
    pred = jnp.min(jnp.where(x == m, cls, _NC), axis=0, keepdims=True)
    a = (cls == t).astype(jnp.bfloat16)
    b = (cls == pred).astype(jnp.bfloat16)
    acc_ref[...] += lax.dot_general(
        a, b, (((1,), (1,)), ((), ())), preferred_element_type=jnp.float32)

    @pl.when(step == num_steps - 1)
    def _write():
        hist_ref[...] = acc_ref[...]


def _tc_partial_hist(logits, tgt):
    steps_per_b = _Q_TC // _TC_CHUNK
    num_steps = _NB * steps_per_b
    return pl.pallas_call(
        functools.partial(_tc_body, num_steps=num_steps, chunk=_TC_CHUNK),
        grid=(num_steps,),
        in_specs=[
            pl.BlockSpec((1, _NC, _TC_CHUNK),
                         lambda i: (i // steps_per_b, 0, i % steps_per_b)),
            pl.BlockSpec((1, 1, _TC_CHUNK),
                         lambda i: (i // steps_per_b, 0, i % steps_per_b)),
        ],
        out_specs=pl.BlockSpec((_NC, _NC), lambda i: (0, 0)),
        out_shape=jax.ShapeDtypeStruct((_NC, _NC), jnp.float32),
        scratch_shapes=[pltpu.VMEM((_NC, _NC), jnp.float32)],
    )(logits, tgt)


def _sc_hist_kernel(logit_hbm, tgt_hbm, out_hbm, buf, tbuf, hist, sem):
    l_sc = _NPIX - _Q_TC
    bpw = l_sc // _NW
    n_chunks = bpw // _SC_CHUNK
    cid = lax.axis_index("c")
    sid = lax.axis_index("s")
    wid = sid * 2 + cid

    zero16 = jnp.zeros((16,), jnp.float32)
    for r in range(0, _NC * 32, 16):
        hist[pl.ds(r, 16)] = zero16

    ones16 = jnp.ones((16,), jnp.float32)
    for b in range(_NB):
        for ch in range(n_chunks):
            q = _Q_TC + wid * bpw + ch * _SC_CHUNK
            copies = []
            for c in range(_NC):
                off = (b * _NC + c) * _NPIX + q
                copies.append(pltpu.async_copy(
                    logit_hbm.at[pl.ds(off, _SC_CHUNK)],
                    buf.at[pl.ds(c * _SC_CHUNK, _SC_CHUNK)], sem))
            tcopy = pltpu.async_copy(
                tgt_hbm.at[pl.ds(b * _NPIX + q, _SC_CHUNK)], tbuf, sem)
            for cp in copies:
                cp.wait()
            tcopy.wait()

            def vec_body(i, _):
                off = i * 16
                m = buf[pl.ds(off, 16)]
                am = jnp.zeros((16,), jnp.int32)
                for c in range(1, _NC):
                    v = buf[pl.ds(c * _SC_CHUNK + off, 16)]
                    gt = v > m
                    m = jnp.where(gt, v, m)
                    am = jnp.where(gt, c, am)
                t = tbuf[pl.ds(off, 16)]
                valid = (t >= 0) & (t < _NC)
                ts = jnp.where(valid, t, 0)
                plsc.addupdate_scatter(hist, [ts * 32 + am], ones16,
                                       mask=valid)
                return 0

            lax.fori_loop(0, _SC_CHUNK // 16, vec_body, 0)

    pltpu.sync_copy(hist, out_hbm.at[pl.ds(wid * _NC * 32, _NC * 32)])


def _sc_partial_hist(logits_flat, tgt_flat):
    mesh = plsc.VectorSubcoreMesh(core_axis_name="c", subcore_axis_name="s")
    f = functools.partial(
        pl.kernel,
        mesh=mesh,
        out_type=jax.ShapeDtypeStruct((_NW * _NC * 32,), jnp.float32),
        scratch_types=[
            pltpu.VMEM((_NC * _SC_CHUNK,), jnp.float32),
            pltpu.VMEM((_SC_CHUNK,), jnp.int32),
            pltpu.VMEM((_NC * 32,), jnp.float32),
            pltpu.SemaphoreType.DMA,
        ],
        compiler_params=pltpu.CompilerParams(needs_layout_passes=False),
    )(_sc_hist_kernel)
    return f(logits_flat, tgt_flat)


def _final_body(tc_ref, sc_ref, o_ref):
    hist = tc_ref[...] + jnp.sum(sc_ref[...], axis=0)[:, :_NC]
    r0 = lax.broadcasted_iota(jnp.int32, (_NC, _NC), 0)
    r1 = lax.broadcasted_iota(jnp.int32, (_NC, _NC), 1)
    diag = (r0 == r1).astype(jnp.float32)
    tp = jnp.sum(hist * diag, axis=1)
    sum1 = jnp.sum(hist, axis=1)
    sum0 = jnp.sum(hist, axis=0)
    iou = tp / (sum1 + sum0 - tp + _EPS)
    o_ref[...] = jnp.reshape(jnp.sum(iou) * (100.0 / _NC), (1, 1))


def _finalize(hist_tc, hist_sc):
    return pl.pallas_call(
        _final_body,
        out_shape=jax.ShapeDtypeStruct((1, 1), jnp.float32),
    )(hist_tc, hist_sc)


def kernel(input_img, input, target):
    del input_img
    n_b, n_c, h, w = input.shape
    logits = input.reshape(n_b, n_c, _NPIX)
    tgt = target.reshape(n_b, 1, _NPIX)
    hist_tc = _tc_partial_hist(logits, tgt)
    hist_sc = _sc_partial_hist(input.reshape(-1), target.reshape(-1))
    return _finalize(hist_tc, hist_sc.reshape(_NW, _NC, 32)).reshape(1)


# speedup vs baseline: 5.2107x; 5.2107x over previous
import functools

import jax
import jax.numpy as jnp
import numpy as np
from jax import lax
from jax.experimental import pallas as pl
from jax.experimental.pallas import tpu as pltpu

_NC = 19
_EPS = float(np.finfo(np.float32).eps)


def _body(x_ref, t_ref, o_ref, acc_ref, *, num_steps, rows):
    step = pl.program_id(0)

    @pl.when(step == 0)
    def _init():
        acc_ref[...] = jnp.zeros_like(acc_ref)

    x = x_ref[0]
    t = t_ref[...]
    cls = lax.broadcasted_iota(jnp.int32, (_NC, rows, 512), 0)
    m = jnp.max(x, axis=0, keepdims=True)
    pred = jnp.min(jnp.where(x == m, cls, _NC), axis=0, keepdims=True)
    npx = rows * 512
    t2 = t.reshape(1, npx)
    p2 = pred.reshape(1, npx)
    cls2 = lax.broadcasted_iota(jnp.int32, (_NC, npx), 0)
    a = (cls2 == t2).astype(jnp.bfloat16)
    b = (cls2 == p2).astype(jnp.bfloat16)
    acc_ref[...] += lax.dot_general(
        a, b, (((1,), (1,)), ((), ())),
        preferred_element_type=jnp.float32)

    @pl.when(step == num_steps - 1)
    def _finalize():
        hist = acc_ref[...]
        r0 = lax.broadcasted_iota(jnp.int32, (_NC, _NC), 0)
        r1 = lax.broadcasted_iota(jnp.int32, (_NC, _NC), 1)
        diag = (r0 == r1).astype(jnp.float32)
        tp = jnp.sum(hist * diag, axis=1)
        sum1 = jnp.sum(hist, axis=1)
        sum0 = jnp.sum(hist, axis=0)
        iou = tp / (sum1 + sum0 - tp + _EPS)
        o_ref[...] = jnp.reshape(jnp.sum(iou) * (100.0 / _NC), (1, 1))


def kernel(input_img, input, target):
    del input_img
    n_b, n_c, h, w = input.shape
    rows = 128
    steps_per_b = h // rows
    num_steps = n_b * steps_per_b

    out = pl.pallas_call(
        functools.partial(_body, num_steps=num_steps, rows=rows),
        grid=(num_steps,),
        in_specs=[
            pl.BlockSpec((1, n_c, rows, w),
                         lambda i: (i // steps_per_b, 0, i % steps_per_b, 0)),
            pl.BlockSpec((1, rows, w),
                         lambda i: (i // steps_per_b, i % steps_per_b, 0)),
        ],
        out_specs=pl.BlockSpec((1, 1), lambda i: (0, 0)),
        out_shape=jax.ShapeDtypeStruct((1, 1), jnp.float32),
        scratch_shapes=[pltpu.VMEM((_NC, _NC), jnp.float32)],
    )(input, target)
    return out.reshape(1)


# speedup vs baseline: 5.2910x; 1.0154x over previous
import functools

import jax
import jax.numpy as jnp
import numpy as np
from jax import lax
from jax.experimental import pallas as pl
from jax.experimental.pallas import tpu as pltpu

_NC = 19
_EPS = float(np.finfo(np.float32).eps)


def _body(x_ref, t_ref, o_ref, acc_ref, *, num_steps, rows):
    step = pl.program_id(0)

    @pl.when(step == 0)
    def _init():
        acc_ref[...] = jnp.zeros_like(acc_ref)

    x = x_ref[0]
    t = t_ref[...]
    cls = lax.broadcasted_iota(jnp.int32, (_NC, rows, 512), 0)
    m = jnp.max(x, axis=0, keepdims=True)
    pred = jnp.min(jnp.where(x == m, cls, _NC), axis=0, keepdims=True)
    npx = rows * 512
    t2 = t.reshape(1, npx)
    p2 = pred.reshape(1, npx)
    cls2 = lax.broadcasted_iota(jnp.int32, (_NC, npx), 0)
    a = (cls2 == t2).astype(jnp.bfloat16)
    b = (cls2 == p2).astype(jnp.bfloat16)
    acc_ref[...] += lax.dot_general(
        a, b, (((1,), (1,)), ((), ())),
        preferred_element_type=jnp.float32)

    @pl.when(step == num_steps - 1)
    def _finalize():
        hist = acc_ref[...]
        r0 = lax.broadcasted_iota(jnp.int32, (_NC, _NC), 0)
        r1 = lax.broadcasted_iota(jnp.int32, (_NC, _NC), 1)
        diag = (r0 == r1).astype(jnp.float32)
        tp = jnp.sum(hist * diag, axis=1)
        sum1 = jnp.sum(hist, axis=1)
        sum0 = jnp.sum(hist, axis=0)
        iou = tp / (sum1 + sum0 - tp + _EPS)
        o_ref[...] = jnp.reshape(jnp.sum(iou) * (100.0 / _NC), (1, 1))


def kernel(input_img, input, target):
    del input_img
    n_b, n_c, h, w = input.shape
    rows = 256
    steps_per_b = h // rows
    num_steps = n_b * steps_per_b

    out = pl.pallas_call(
        functools.partial(_body, num_steps=num_steps, rows=rows),
        grid=(num_steps,),
        in_specs=[
            pl.BlockSpec((1, n_c, rows, w),
                         lambda i: (i // steps_per_b, 0, i % steps_per_b, 0)),
            pl.BlockSpec((1, rows, w),
                         lambda i: (i // steps_per_b, i % steps_per_b, 0)),
        ],
        out_specs=pl.BlockSpec((1, 1), lambda i: (0, 0)),
        out_shape=jax.ShapeDtypeStruct((1, 1), jnp.float32),
        scratch_shapes=[pltpu.VMEM((_NC, _NC), jnp.float32)],
    )(input, target)
    return out.reshape(1)
